# baseline (speedup 1.0000x reference)
import jax
import jax.numpy as jnp
from jax import lax
from jax.experimental import pallas as pl
from jax.experimental.pallas import tpu as pltpu
from jax.experimental.pallas import tpu_sc as plsc

_R = 131072
_W = 128
_NC = 2
_NS = 16
_NW = _NC * _NS
_CHUNK = 128
_CHUNKS_PER_WORKER = _R // (_NW * _CHUNK)


def _sc_flip(x_hbm, o_hbm, idx_v, rows_v, sem):
    wid = lax.axis_index("s") * _NC + lax.axis_index("c")
    base_chunk = wid * _CHUNKS_PER_WORKER
    lane = lax.iota(jnp.int32, 16)

    def body(c, carry):
        slab = base_chunk + c
        top = slab * _CHUNK + (_CHUNK - 1)
        for j in range(_CHUNK // 16):
            idx_v[pl.ds(j * 16, 16)] = top - j * 16 - lane
        pltpu.async_copy(x_hbm.at[idx_v], rows_v, sem).wait()
        pltpu.sync_copy(rows_v, o_hbm.at[pl.ds(slab * _CHUNK, _CHUNK)])
        return carry

    lax.fori_loop(0, _CHUNKS_PER_WORKER, body, None)


def kernel(x):
    B, C, D, H, W = x.shape
    xr = x.reshape(B * C * D * H, W)
    mesh = plsc.VectorSubcoreMesh(core_axis_name="c", subcore_axis_name="s")
    k = pl.kernel(
        _sc_flip,
        mesh=mesh,
        out_type=jax.ShapeDtypeStruct((_R, _W), jnp.float32),
        scratch_types=[
            pltpu.VMEM((_CHUNK,), jnp.int32),
            pltpu.VMEM((_CHUNK, _W), jnp.float32),
            pltpu.SemaphoreType.DMA,
        ],
    )
    out = k(xr)
    return out.reshape(B, C, D, H, W)


# speedup vs baseline: 1.1695x; 1.1695x over previous
import jax
import jax.numpy as jnp
from jax.experimental import pallas as pl

_HB = 8


def _flip_body(x_ref, o_ref):
    x = x_ref[...]
    o_ref[...] = jnp.concatenate(
        [x[:, i : i + 1, :] for i in reversed(range(_HB))], axis=1
    )


def kernel(x):
    B, C, D, H, W = x.shape
    L = B * C * D
    xr = x.reshape(L, H, W)
    Lb = 256
    nH = H // _HB
    out = pl.pallas_call(
        _flip_body,
        grid=(L // Lb, nH),
        in_specs=[pl.BlockSpec((Lb, _HB, W), lambda l, h: (l, h, 0))],
        out_specs=pl.BlockSpec((Lb, _HB, W), lambda l, h: (l, nH - 1 - h, 0)),
        out_shape=jax.ShapeDtypeStruct((L, H, W), x.dtype),
    )(xr)
    return out.reshape(B, C, D, H, W)
